# baseline (speedup 1.0000x reference)
import functools

import jax
import jax.numpy as jnp
from jax import lax
from jax.experimental import pallas as pl
from jax.experimental.pallas import tpu as pltpu
from jax.experimental.pallas import tpu_sc as plsc

N = 10000
E = 320000
D_IN = 128
D_H = 256
HALF = 128
NC, NS, L = 2, 16, 16
CH = 32
NBUF = 8
IDXBLK = 64
DEGW = 8
NPAD = 10112
EPAD = 327680
ZSTRIPE = NPAD // NS
EDGES_PER_TILE = EPAD // NS
CHUNKS_AGG = EDGES_PER_TILE // CH
EDGES_PER_WORKER = EPAD // (NS * NC)
CHUNKS_DEG = EDGES_PER_WORKER // CH
RB = 2000
GRID = N // RB


def _sc_mesh():
    return plsc.VectorSubcoreMesh(
        core_axis_name="c", subcore_axis_name="s", num_cores=NC, num_subcores=NS
    )


@functools.partial(
    pl.kernel,
    out_type=jax.ShapeDtypeStruct((NC * NPAD, HALF), jnp.float32),
    mesh=_sc_mesh(),
    scratch_types=[
        pltpu.VMEM((CHUNKS_DEG, CH), jnp.int32),
        pltpu.VMEM((CH, HALF), jnp.float32),
        pltpu.VMEM_SHARED((NPAD, HALF), jnp.float32),
        pltpu.SemaphoreType.DMA,
    ],
)
def _deg_kernel(dst_hbm, ones_hbm, zeros_hbm, out_hbm, didx_v, ones_v, acc_sh, sem):
    cid = lax.axis_index("c")
    sid = lax.axis_index("s")
    pltpu.sync_copy(
        zeros_hbm.at[pl.ds(sid * ZSTRIPE, ZSTRIPE)],
        acc_sh.at[pl.ds(sid * ZSTRIPE, ZSTRIPE)],
    )
    pltpu.sync_copy(ones_hbm, ones_v)
    wid = sid * NC + cid
    pltpu.sync_copy(dst_hbm.at[pl.ds(wid * CHUNKS_DEG, CHUNKS_DEG)], didx_v)
    plsc.subcore_barrier()

    def body(k, carry):
        @pl.when(k >= DEGW)
        def _():
            pltpu.make_async_copy(ones_v, acc_sh.at[didx_v.at[0]], sem).wait()

        pltpu.async_copy(ones_v, acc_sh.at[didx_v.at[k]], sem, add=True)
        return carry

    lax.fori_loop(0, CHUNKS_DEG, body, 0)

    def drain(k, carry):
        pltpu.make_async_copy(ones_v, acc_sh.at[didx_v.at[0]], sem).wait()
        return carry

    lax.fori_loop(0, DEGW, drain, 0)
    plsc.subcore_barrier()
    pltpu.sync_copy(
        acc_sh.at[pl.ds(sid * ZSTRIPE, ZSTRIPE)],
        out_hbm.at[pl.ds(cid * NPAD + sid * ZSTRIPE, ZSTRIPE)],
    )


def _agg_pipeline(table_hbm, gidx_hbm, dst_hbm, out_half_hbm, sid,
                  gidx_v, didx_v, rows_v, acc_sh, gsems, ssems):
    base = sid * CHUNKS_AGG

    def block(blk, carry):
        pltpu.sync_copy(gidx_hbm.at[pl.ds(base + blk * IDXBLK, IDXBLK)], gidx_v)
        pltpu.sync_copy(dst_hbm.at[pl.ds(base + blk * IDXBLK, IDXBLK)], didx_v)

        def group(g, c):
            for b in range(NBUF):
                k = g * NBUF + b

                @pl.when(g > 0)
                def _(b=b, k=k):
                    pltpu.make_async_copy(
                        rows_v.at[b], acc_sh.at[didx_v.at[k - NBUF]], ssems[b]
                    ).wait()

                pltpu.async_copy(table_hbm.at[gidx_v.at[k]], rows_v.at[b],
                                 gsems[b])
            for b in range(NBUF):
                k = g * NBUF + b
                pltpu.make_async_copy(
                    table_hbm.at[gidx_v.at[k]], rows_v.at[b], gsems[b]).wait()
                pltpu.async_copy(
                    rows_v.at[b], acc_sh.at[didx_v.at[k]], ssems[b], add=True)
            return c

        lax.fori_loop(0, IDXBLK // NBUF, group, 0)
        for b in range(NBUF):
            pltpu.make_async_copy(
                rows_v.at[b], acc_sh.at[didx_v.at[b]], ssems[b]).wait()
        return carry

    lax.fori_loop(0, CHUNKS_AGG // IDXBLK, block, 0)
    plsc.subcore_barrier()
    pltpu.sync_copy(
        acc_sh.at[pl.ds(sid * ZSTRIPE, ZSTRIPE)],
        out_half_hbm.at[pl.ds(sid * ZSTRIPE, ZSTRIPE)],
    )


@functools.partial(
    pl.kernel,
    out_type=[
        jax.ShapeDtypeStruct((NPAD, HALF), jnp.float32),
        jax.ShapeDtypeStruct((NPAD, HALF), jnp.float32),
    ],
    mesh=_sc_mesh(),
    scratch_types=[
        pltpu.VMEM((IDXBLK, CH), jnp.int32),
        pltpu.VMEM((IDXBLK, CH), jnp.int32),
        pltpu.VMEM((NBUF, CH, HALF), jnp.float32),
        pltpu.VMEM_SHARED((NPAD, HALF), jnp.float32),
        [pltpu.SemaphoreType.DMA] * NBUF,
        [pltpu.SemaphoreType.DMA] * NBUF,
    ],
)
def _agg_kernel(ylo_hbm, yhi_hbm, gidx_hbm, dst_hbm, zeros_hbm,
                outlo_hbm, outhi_hbm,
                gidx_v, didx_v, rows_v, acc_sh, gsems, ssems):
    cid = lax.axis_index("c")
    sid = lax.axis_index("s")
    pltpu.sync_copy(
        zeros_hbm.at[pl.ds(sid * ZSTRIPE, ZSTRIPE)],
        acc_sh.at[pl.ds(sid * ZSTRIPE, ZSTRIPE)],
    )
    plsc.subcore_barrier()

    @pl.when(cid == 0)
    def _():
        _agg_pipeline(ylo_hbm, gidx_hbm, dst_hbm, outlo_hbm, sid,
                      gidx_v, didx_v, rows_v, acc_sh, gsems, ssems)

    @pl.when(cid == 1)
    def _():
        _agg_pipeline(yhi_hbm, gidx_hbm, dst_hbm, outhi_hbm, sid,
                      gidx_v, didx_v, rows_v, acc_sh, gsems, ssems)


def _b1_body(x_ref, w_ref, degp_ref, ylo_ref, yhi_ref, dinv_ref):
    deg = degp_ref[0][:, 0:1] + degp_ref[1][:, 0:1] + 1.0
    dinv = lax.rsqrt(deg)
    xw = jnp.dot(x_ref[...], w_ref[...], preferred_element_type=jnp.float32)
    y = xw * dinv
    ylo_ref[...] = y[:, :HALF]
    yhi_ref[...] = y[:, HALF:]
    dinv_ref[...] = dinv


def _first_layer(x, W1, degp):
    return pl.pallas_call(
        _b1_body,
        grid=(GRID,),
        in_specs=[
            pl.BlockSpec((RB, D_IN), lambda i: (i, 0)),
            pl.BlockSpec((D_IN, D_H), lambda i: (0, 0)),
            pl.BlockSpec((2, RB, HALF), lambda i: (0, i, 0)),
        ],
        out_specs=[
            pl.BlockSpec((RB, HALF), lambda i: (i, 0)),
            pl.BlockSpec((RB, HALF), lambda i: (i, 0)),
            pl.BlockSpec((RB, 1), lambda i: (i, 0)),
        ],
        out_shape=[
            jax.ShapeDtypeStruct((N, HALF), jnp.float32),
            jax.ShapeDtypeStruct((N, HALF), jnp.float32),
            jax.ShapeDtypeStruct((N, 1), jnp.float32),
        ],
    )(x, W1, degp)


def _mid_body(alo_ref, ahi_ref, ylo_ref, yhi_ref, dinv_ref, w_ref, b_ref,
              olo_ref, ohi_ref):
    dinv = dinv_ref[...]
    agg = jnp.concatenate([alo_ref[...] + ylo_ref[...],
                           ahi_ref[...] + yhi_ref[...]], axis=1)
    h = jnp.maximum(agg * dinv + b_ref[...], 0.0)
    y = jnp.dot(h, w_ref[...], preferred_element_type=jnp.float32) * dinv
    olo_ref[...] = y[:, :HALF]
    ohi_ref[...] = y[:, HALF:]


def _mid_layer(alo, ahi, ylo, yhi, dinv, W, b):
    return pl.pallas_call(
        _mid_body,
        grid=(GRID,),
        in_specs=[
            pl.BlockSpec((RB, HALF), lambda i: (i, 0)),
            pl.BlockSpec((RB, HALF), lambda i: (i, 0)),
            pl.BlockSpec((RB, HALF), lambda i: (i, 0)),
            pl.BlockSpec((RB, HALF), lambda i: (i, 0)),
            pl.BlockSpec((RB, 1), lambda i: (i, 0)),
            pl.BlockSpec((D_H, D_H), lambda i: (0, 0)),
            pl.BlockSpec((1, D_H), lambda i: (0, 0)),
        ],
        out_specs=[
            pl.BlockSpec((RB, HALF), lambda i: (i, 0)),
            pl.BlockSpec((RB, HALF), lambda i: (i, 0)),
        ],
        out_shape=[
            jax.ShapeDtypeStruct((N, HALF), jnp.float32),
            jax.ShapeDtypeStruct((N, HALF), jnp.float32),
        ],
    )(alo, ahi, ylo, yhi, dinv, W, b.reshape(1, D_H))


def _head_body(alo_ref, ahi_ref, ylo_ref, yhi_ref, dinv_ref, b3_ref,
               fc1w_ref, fc1b_ref, fc2w_ref, fc2b_ref, out_ref):
    dinv = dinv_ref[...]
    agg = jnp.concatenate([alo_ref[...] + ylo_ref[...],
                           ahi_ref[...] + yhi_ref[...]], axis=1)
    h = jnp.maximum(agg * dinv + b3_ref[...], 0.0)
    z = jnp.maximum(
        jnp.dot(h, fc1w_ref[...], preferred_element_type=jnp.float32)
        + fc1b_ref[...], 0.0)
    out_ref[...] = (
        jnp.dot(z, fc2w_ref[...], preferred_element_type=jnp.float32)
        + fc2b_ref[...])


def _head(alo, ahi, ylo, yhi, dinv, b3, fc1_w, fc1_b, fc2_w, fc2_b):
    return pl.pallas_call(
        _head_body,
        grid=(GRID,),
        in_specs=[
            pl.BlockSpec((RB, HALF), lambda i: (i, 0)),
            pl.BlockSpec((RB, HALF), lambda i: (i, 0)),
            pl.BlockSpec((RB, HALF), lambda i: (i, 0)),
            pl.BlockSpec((RB, HALF), lambda i: (i, 0)),
            pl.BlockSpec((RB, 1), lambda i: (i, 0)),
            pl.BlockSpec((1, D_H), lambda i: (0, 0)),
            pl.BlockSpec((D_H, D_H // 2), lambda i: (0, 0)),
            pl.BlockSpec((1, D_H // 2), lambda i: (0, 0)),
            pl.BlockSpec((D_H // 2, 1), lambda i: (0, 0)),
            pl.BlockSpec((1, 1), lambda i: (0, 0)),
        ],
        out_specs=pl.BlockSpec((RB, 1), lambda i: (i, 0)),
        out_shape=jax.ShapeDtypeStruct((N, 1), jnp.float32),
    )(alo, ahi, ylo, yhi, dinv, b3.reshape(1, D_H), fc1_w,
      fc1_b.reshape(1, D_H // 2), fc2_w, fc2_b.reshape(1, 1))


def kernel(x, edge_index, W1, b1, W2, b2, W3, b3, fc1_w, fc1_b, fc2_w, fc2_b):
    src = edge_index[0].astype(jnp.int32)
    dst = edge_index[1].astype(jnp.int32)
    npad = EPAD - E
    pad_idx = jnp.arange(npad, dtype=jnp.int32)
    src_p = jnp.concatenate([src, pad_idx % N])
    dst_p = jnp.concatenate([dst, N + (pad_idx % L)])
    gidx = src_p.reshape(EPAD // CH, CH)
    dst2d = dst_p.reshape(EPAD // CH, CH)

    ones_deg = jnp.ones((CH, HALF), jnp.float32)
    zeros_agg = jnp.zeros((NPAD, HALF), jnp.float32)

    degp = _deg_kernel(dst2d, ones_deg, zeros_agg)
    degp = degp.reshape(NC, NPAD, HALF)

    y1lo, y1hi, dinv = _first_layer(x, W1, degp)
    a1lo, a1hi = _agg_kernel(y1lo, y1hi, gidx, dst2d, zeros_agg)
    y2lo, y2hi = _mid_layer(a1lo, a1hi, y1lo, y1hi, dinv, W2, b1)
    a2lo, a2hi = _agg_kernel(y2lo, y2hi, gidx, dst2d, zeros_agg)
    y3lo, y3hi = _mid_layer(a2lo, a2hi, y2lo, y2hi, dinv, W3, b2)
    a3lo, a3hi = _agg_kernel(y3lo, y3hi, gidx, dst2d, zeros_agg)
    out = _head(a3lo, a3hi, y3lo, y3hi, dinv, b3,
                fc1_w, fc1_b, fc2_w, fc2_b)
    return out


# speedup vs baseline: 1.0304x; 1.0304x over previous
import functools

import jax
import jax.numpy as jnp
from jax import lax
from jax.experimental import pallas as pl
from jax.experimental.pallas import tpu as pltpu
from jax.experimental.pallas import tpu_sc as plsc

N = 10000
E = 320000
D_IN = 128
D_H = 256
HALF = 128
NC, NS, L = 2, 16, 16
CH = 64
NBUF = 4
IDXBLK = 64
DEGW = 8
NPAD = 10112
EPAD = 327680
ZSTRIPE = NPAD // NS
EDGES_PER_TILE = EPAD // NS
CHUNKS_AGG = EDGES_PER_TILE // CH
EDGES_PER_WORKER = EPAD // (NS * NC)
CHUNKS_DEG = EDGES_PER_WORKER // CH
RB = 2000
GRID = N // RB


def _sc_mesh():
    return plsc.VectorSubcoreMesh(
        core_axis_name="c", subcore_axis_name="s", num_cores=NC, num_subcores=NS
    )


@functools.partial(
    pl.kernel,
    out_type=jax.ShapeDtypeStruct((NC * NPAD, HALF), jnp.float32),
    mesh=_sc_mesh(),
    scratch_types=[
        pltpu.VMEM((CHUNKS_DEG, CH), jnp.int32),
        pltpu.VMEM((CH, HALF), jnp.float32),
        pltpu.VMEM_SHARED((NPAD, HALF), jnp.float32),
        pltpu.SemaphoreType.DMA,
    ],
)
def _deg_kernel(dst_hbm, ones_hbm, zeros_hbm, out_hbm, didx_v, ones_v, acc_sh, sem):
    cid = lax.axis_index("c")
    sid = lax.axis_index("s")
    pltpu.sync_copy(
        zeros_hbm.at[pl.ds(sid * ZSTRIPE, ZSTRIPE)],
        acc_sh.at[pl.ds(sid * ZSTRIPE, ZSTRIPE)],
    )
    pltpu.sync_copy(ones_hbm, ones_v)
    wid = sid * NC + cid
    pltpu.sync_copy(dst_hbm.at[pl.ds(wid * CHUNKS_DEG, CHUNKS_DEG)], didx_v)
    plsc.subcore_barrier()

    def body(k, carry):
        @pl.when(k >= DEGW)
        def _():
            pltpu.make_async_copy(ones_v, acc_sh.at[didx_v.at[0]], sem).wait()

        pltpu.async_copy(ones_v, acc_sh.at[didx_v.at[k]], sem, add=True)
        return carry

    lax.fori_loop(0, CHUNKS_DEG, body, 0)

    def drain(k, carry):
        pltpu.make_async_copy(ones_v, acc_sh.at[didx_v.at[0]], sem).wait()
        return carry

    lax.fori_loop(0, DEGW, drain, 0)
    plsc.subcore_barrier()
    pltpu.sync_copy(
        acc_sh.at[pl.ds(sid * ZSTRIPE, ZSTRIPE)],
        out_hbm.at[pl.ds(cid * NPAD + sid * ZSTRIPE, ZSTRIPE)],
    )


def _agg_pipeline(table_hbm, gidx_hbm, dst_hbm, out_half_hbm, sid,
                  gidx_v, didx_v, rows_v, acc_sh, gsems, ssems):
    base = sid * CHUNKS_AGG

    def block(blk, carry):
        pltpu.sync_copy(gidx_hbm.at[pl.ds(base + blk * IDXBLK, IDXBLK)], gidx_v)
        pltpu.sync_copy(dst_hbm.at[pl.ds(base + blk * IDXBLK, IDXBLK)], didx_v)

        def group(g, c):
            for b in range(NBUF):
                k = g * NBUF + b

                @pl.when(g > 0)
                def _(b=b, k=k):
                    pltpu.make_async_copy(
                        rows_v.at[b], acc_sh.at[didx_v.at[k - NBUF]], ssems[b]
                    ).wait()

                pltpu.async_copy(table_hbm.at[gidx_v.at[k]], rows_v.at[b],
                                 gsems[b])
            for b in range(NBUF):
                k = g * NBUF + b
                pltpu.make_async_copy(
                    table_hbm.at[gidx_v.at[k]], rows_v.at[b], gsems[b]).wait()
                pltpu.async_copy(
                    rows_v.at[b], acc_sh.at[didx_v.at[k]], ssems[b], add=True)
            return c

        lax.fori_loop(0, IDXBLK // NBUF, group, 0)
        for b in range(NBUF):
            pltpu.make_async_copy(
                rows_v.at[b], acc_sh.at[didx_v.at[b]], ssems[b]).wait()
        return carry

    lax.fori_loop(0, CHUNKS_AGG // IDXBLK, block, 0)
    plsc.subcore_barrier()
    pltpu.sync_copy(
        acc_sh.at[pl.ds(sid * ZSTRIPE, ZSTRIPE)],
        out_half_hbm.at[pl.ds(sid * ZSTRIPE, ZSTRIPE)],
    )


@functools.partial(
    pl.kernel,
    out_type=[
        jax.ShapeDtypeStruct((NPAD, HALF), jnp.float32),
        jax.ShapeDtypeStruct((NPAD, HALF), jnp.float32),
    ],
    mesh=_sc_mesh(),
    scratch_types=[
        pltpu.VMEM((IDXBLK, CH), jnp.int32),
        pltpu.VMEM((IDXBLK, CH), jnp.int32),
        pltpu.VMEM((NBUF, CH, HALF), jnp.float32),
        pltpu.VMEM_SHARED((NPAD, HALF), jnp.float32),
        [pltpu.SemaphoreType.DMA] * NBUF,
        [pltpu.SemaphoreType.DMA] * NBUF,
    ],
)
def _agg_kernel(ylo_hbm, yhi_hbm, gidx_hbm, dst_hbm, zeros_hbm,
                outlo_hbm, outhi_hbm,
                gidx_v, didx_v, rows_v, acc_sh, gsems, ssems):
    cid = lax.axis_index("c")
    sid = lax.axis_index("s")
    pltpu.sync_copy(
        zeros_hbm.at[pl.ds(sid * ZSTRIPE, ZSTRIPE)],
        acc_sh.at[pl.ds(sid * ZSTRIPE, ZSTRIPE)],
    )
    plsc.subcore_barrier()

    @pl.when(cid == 0)
    def _():
        _agg_pipeline(ylo_hbm, gidx_hbm, dst_hbm, outlo_hbm, sid,
                      gidx_v, didx_v, rows_v, acc_sh, gsems, ssems)

    @pl.when(cid == 1)
    def _():
        _agg_pipeline(yhi_hbm, gidx_hbm, dst_hbm, outhi_hbm, sid,
                      gidx_v, didx_v, rows_v, acc_sh, gsems, ssems)


def _b1_body(x_ref, w_ref, degp_ref, ylo_ref, yhi_ref, dinv_ref):
    deg = degp_ref[0][:, 0:1] + degp_ref[1][:, 0:1] + 1.0
    dinv = lax.rsqrt(deg)
    xw = jnp.dot(x_ref[...], w_ref[...], preferred_element_type=jnp.float32)
    y = xw * dinv
    ylo_ref[...] = y[:, :HALF]
    yhi_ref[...] = y[:, HALF:]
    dinv_ref[...] = dinv


def _first_layer(x, W1, degp):
    return pl.pallas_call(
        _b1_body,
        grid=(GRID,),
        in_specs=[
            pl.BlockSpec((RB, D_IN), lambda i: (i, 0)),
            pl.BlockSpec((D_IN, D_H), lambda i: (0, 0)),
            pl.BlockSpec((2, RB, HALF), lambda i: (0, i, 0)),
        ],
        out_specs=[
            pl.BlockSpec((RB, HALF), lambda i: (i, 0)),
            pl.BlockSpec((RB, HALF), lambda i: (i, 0)),
            pl.BlockSpec((RB, 1), lambda i: (i, 0)),
        ],
        out_shape=[
            jax.ShapeDtypeStruct((N, HALF), jnp.float32),
            jax.ShapeDtypeStruct((N, HALF), jnp.float32),
            jax.ShapeDtypeStruct((N, 1), jnp.float32),
        ],
    )(x, W1, degp)


def _mid_body(alo_ref, ahi_ref, ylo_ref, yhi_ref, dinv_ref, w_ref, b_ref,
              olo_ref, ohi_ref):
    dinv = dinv_ref[...]
    agg = jnp.concatenate([alo_ref[...] + ylo_ref[...],
                           ahi_ref[...] + yhi_ref[...]], axis=1)
    h = jnp.maximum(agg * dinv + b_ref[...], 0.0)
    y = jnp.dot(h, w_ref[...], preferred_element_type=jnp.float32) * dinv
    olo_ref[...] = y[:, :HALF]
    ohi_ref[...] = y[:, HALF:]


def _mid_layer(alo, ahi, ylo, yhi, dinv, W, b):
    return pl.pallas_call(
        _mid_body,
        grid=(GRID,),
        in_specs=[
            pl.BlockSpec((RB, HALF), lambda i: (i, 0)),
            pl.BlockSpec((RB, HALF), lambda i: (i, 0)),
            pl.BlockSpec((RB, HALF), lambda i: (i, 0)),
            pl.BlockSpec((RB, HALF), lambda i: (i, 0)),
            pl.BlockSpec((RB, 1), lambda i: (i, 0)),
            pl.BlockSpec((D_H, D_H), lambda i: (0, 0)),
            pl.BlockSpec((1, D_H), lambda i: (0, 0)),
        ],
        out_specs=[
            pl.BlockSpec((RB, HALF), lambda i: (i, 0)),
            pl.BlockSpec((RB, HALF), lambda i: (i, 0)),
        ],
        out_shape=[
            jax.ShapeDtypeStruct((N, HALF), jnp.float32),
            jax.ShapeDtypeStruct((N, HALF), jnp.float32),
        ],
    )(alo, ahi, ylo, yhi, dinv, W, b.reshape(1, D_H))


def _head_body(alo_ref, ahi_ref, ylo_ref, yhi_ref, dinv_ref, b3_ref,
               fc1w_ref, fc1b_ref, fc2w_ref, fc2b_ref, out_ref):
    dinv = dinv_ref[...]
    agg = jnp.concatenate([alo_ref[...] + ylo_ref[...],
                           ahi_ref[...] + yhi_ref[...]], axis=1)
    h = jnp.maximum(agg * dinv + b3_ref[...], 0.0)
    z = jnp.maximum(
        jnp.dot(h, fc1w_ref[...], preferred_element_type=jnp.float32)
        + fc1b_ref[...], 0.0)
    out_ref[...] = (
        jnp.dot(z, fc2w_ref[...], preferred_element_type=jnp.float32)
        + fc2b_ref[...])


def _head(alo, ahi, ylo, yhi, dinv, b3, fc1_w, fc1_b, fc2_w, fc2_b):
    return pl.pallas_call(
        _head_body,
        grid=(GRID,),
        in_specs=[
            pl.BlockSpec((RB, HALF), lambda i: (i, 0)),
            pl.BlockSpec((RB, HALF), lambda i: (i, 0)),
            pl.BlockSpec((RB, HALF), lambda i: (i, 0)),
            pl.BlockSpec((RB, HALF), lambda i: (i, 0)),
            pl.BlockSpec((RB, 1), lambda i: (i, 0)),
            pl.BlockSpec((1, D_H), lambda i: (0, 0)),
            pl.BlockSpec((D_H, D_H // 2), lambda i: (0, 0)),
            pl.BlockSpec((1, D_H // 2), lambda i: (0, 0)),
            pl.BlockSpec((D_H // 2, 1), lambda i: (0, 0)),
            pl.BlockSpec((1, 1), lambda i: (0, 0)),
        ],
        out_specs=pl.BlockSpec((RB, 1), lambda i: (i, 0)),
        out_shape=jax.ShapeDtypeStruct((N, 1), jnp.float32),
    )(alo, ahi, ylo, yhi, dinv, b3.reshape(1, D_H), fc1_w,
      fc1_b.reshape(1, D_H // 2), fc2_w, fc2_b.reshape(1, 1))


def kernel(x, edge_index, W1, b1, W2, b2, W3, b3, fc1_w, fc1_b, fc2_w, fc2_b):
    src = edge_index[0].astype(jnp.int32)
    dst = edge_index[1].astype(jnp.int32)
    npad = EPAD - E
    pad_idx = jnp.arange(npad, dtype=jnp.int32)
    src_p = jnp.concatenate([src, pad_idx % N])
    dst_p = jnp.concatenate([dst, N + (pad_idx % L)])
    gidx = src_p.reshape(EPAD // CH, CH)
    dst2d = dst_p.reshape(EPAD // CH, CH)

    ones_deg = jnp.ones((CH, HALF), jnp.float32)
    zeros_agg = jnp.zeros((NPAD, HALF), jnp.float32)

    degp = _deg_kernel(dst2d, ones_deg, zeros_agg)
    degp = degp.reshape(NC, NPAD, HALF)

    y1lo, y1hi, dinv = _first_layer(x, W1, degp)
    a1lo, a1hi = _agg_kernel(y1lo, y1hi, gidx, dst2d, zeros_agg)
    y2lo, y2hi = _mid_layer(a1lo, a1hi, y1lo, y1hi, dinv, W2, b1)
    a2lo, a2hi = _agg_kernel(y2lo, y2hi, gidx, dst2d, zeros_agg)
    y3lo, y3hi = _mid_layer(a2lo, a2hi, y2lo, y2hi, dinv, W3, b2)
    a3lo, a3hi = _agg_kernel(y3lo, y3hi, gidx, dst2d, zeros_agg)
    out = _head(a3lo, a3hi, y3lo, y3hi, dinv, b3,
                fc1_w, fc1_b, fc2_w, fc2_b)
    return out


# speedup vs baseline: 1.0394x; 1.0088x over previous
import functools

import jax
import jax.numpy as jnp
from jax import lax
from jax.experimental import pallas as pl
from jax.experimental.pallas import tpu as pltpu
from jax.experimental.pallas import tpu_sc as plsc

N = 10000
E = 320000
D_IN = 128
D_H = 256
HALF = 128
NC, NS, L = 2, 16, 16
CH = 64
NBUF = 4
IDXBLK = 64
DEGW = 8
NPAD = 10112
EPAD = 327680
ZSTRIPE = NPAD // NS
EDGES_PER_TILE = EPAD // NS
CHUNKS_AGG = EDGES_PER_TILE // CH
EDGES_PER_WORKER = EPAD // (NS * NC)
CHUNKS_DEG = EDGES_PER_WORKER // CH
RB = 2000
GRID = N // RB


def _sc_mesh():
    return plsc.VectorSubcoreMesh(
        core_axis_name="c", subcore_axis_name="s", num_cores=NC, num_subcores=NS
    )


@functools.partial(
    pl.kernel,
    out_type=jax.ShapeDtypeStruct((NC * NPAD, HALF), jnp.float32),
    mesh=_sc_mesh(),
    scratch_types=[
        pltpu.VMEM((CHUNKS_DEG, CH), jnp.int32),
        pltpu.VMEM((CH, HALF), jnp.float32),
        pltpu.VMEM_SHARED((NPAD, HALF), jnp.float32),
        pltpu.SemaphoreType.DMA,
    ],
)
def _deg_kernel(dst_hbm, ones_hbm, zeros_hbm, out_hbm, didx_v, ones_v, acc_sh, sem):
    cid = lax.axis_index("c")
    sid = lax.axis_index("s")
    pltpu.sync_copy(
        zeros_hbm.at[pl.ds(sid * ZSTRIPE, ZSTRIPE)],
        acc_sh.at[pl.ds(sid * ZSTRIPE, ZSTRIPE)],
    )
    pltpu.sync_copy(ones_hbm, ones_v)
    wid = sid * NC + cid
    pltpu.sync_copy(dst_hbm.at[pl.ds(wid * CHUNKS_DEG, CHUNKS_DEG)], didx_v)
    plsc.subcore_barrier()

    def body(k, carry):
        @pl.when(k >= DEGW)
        def _():
            pltpu.make_async_copy(ones_v, acc_sh.at[didx_v.at[0]], sem).wait()

        pltpu.async_copy(ones_v, acc_sh.at[didx_v.at[k]], sem, add=True)
        return carry

    lax.fori_loop(0, CHUNKS_DEG, body, 0)

    def drain(k, carry):
        pltpu.make_async_copy(ones_v, acc_sh.at[didx_v.at[0]], sem).wait()
        return carry

    lax.fori_loop(0, DEGW, drain, 0)
    plsc.subcore_barrier()
    pltpu.sync_copy(
        acc_sh.at[pl.ds(sid * ZSTRIPE, ZSTRIPE)],
        out_hbm.at[pl.ds(cid * NPAD + sid * ZSTRIPE, ZSTRIPE)],
    )


def _agg_pipeline(table_hbm, gidx_hbm, dst_hbm, out_half_hbm, sid,
                  gidx_v, didx_v, rows_v, acc_sh, gsems, ssems):
    base = sid * CHUNKS_AGG

    def block(blk, carry):
        pltpu.sync_copy(gidx_hbm.at[pl.ds(base + blk * IDXBLK, IDXBLK)], gidx_v)
        pltpu.sync_copy(dst_hbm.at[pl.ds(base + blk * IDXBLK, IDXBLK)], didx_v)

        def group(g, c):
            for b in range(NBUF):
                k = g * NBUF + b

                @pl.when(g > 0)
                def _(b=b, k=k):
                    pltpu.make_async_copy(
                        rows_v.at[b], acc_sh.at[didx_v.at[k - NBUF]], ssems[b]
                    ).wait()

                pltpu.async_copy(table_hbm.at[gidx_v.at[k]], rows_v.at[b],
                                 gsems[b])
            for b in range(NBUF):
                k = g * NBUF + b
                pltpu.make_async_copy(
                    table_hbm.at[gidx_v.at[k]], rows_v.at[b], gsems[b]).wait()
                pltpu.async_copy(
                    rows_v.at[b], acc_sh.at[didx_v.at[k]], ssems[b], add=True)
            return c

        lax.fori_loop(0, IDXBLK // NBUF, group, 0)
        for b in range(NBUF):
            pltpu.make_async_copy(
                rows_v.at[b], acc_sh.at[didx_v.at[b]], ssems[b]).wait()
        return carry

    lax.fori_loop(0, CHUNKS_AGG // IDXBLK, block, 0)
    plsc.subcore_barrier()
    pltpu.sync_copy(
        acc_sh.at[pl.ds(sid * ZSTRIPE, ZSTRIPE)],
        out_half_hbm.at[pl.ds(sid * ZSTRIPE, ZSTRIPE)],
    )


@functools.partial(
    pl.kernel,
    out_type=[
        jax.ShapeDtypeStruct((NPAD, HALF), jnp.float32),
        jax.ShapeDtypeStruct((NPAD, HALF), jnp.float32),
    ],
    mesh=_sc_mesh(),
    scratch_types=[
        pltpu.VMEM((IDXBLK, CH), jnp.int32),
        pltpu.VMEM((IDXBLK, CH), jnp.int32),
        pltpu.VMEM((NBUF, CH, HALF), jnp.float32),
        pltpu.VMEM_SHARED((NPAD, HALF), jnp.float32),
        [pltpu.SemaphoreType.DMA] * NBUF,
        [pltpu.SemaphoreType.DMA] * NBUF,
    ],
)
def _agg_kernel(ylo_hbm, yhi_hbm, gidx_hbm, dst_hbm, zeros_hbm,
                outlo_hbm, outhi_hbm,
                gidx_v, didx_v, rows_v, acc_sh, gsems, ssems):
    cid = lax.axis_index("c")
    sid = lax.axis_index("s")
    pltpu.sync_copy(
        zeros_hbm.at[pl.ds(sid * ZSTRIPE, ZSTRIPE)],
        acc_sh.at[pl.ds(sid * ZSTRIPE, ZSTRIPE)],
    )
    plsc.subcore_barrier()

    @pl.when(cid == 0)
    def _():
        _agg_pipeline(ylo_hbm, gidx_hbm, dst_hbm, outlo_hbm, sid,
                      gidx_v, didx_v, rows_v, acc_sh, gsems, ssems)

    @pl.when(cid == 1)
    def _():
        _agg_pipeline(yhi_hbm, gidx_hbm, dst_hbm, outhi_hbm, sid,
                      gidx_v, didx_v, rows_v, acc_sh, gsems, ssems)


def _xw_body(x_ref, w_ref, xw_ref):
    xw_ref[...] = jnp.dot(x_ref[...], w_ref[...],
                          preferred_element_type=jnp.float32)


def _xw_first(x, W1):
    return pl.pallas_call(
        _xw_body,
        grid=(GRID,),
        in_specs=[
            pl.BlockSpec((RB, D_IN), lambda i: (i, 0)),
            pl.BlockSpec((D_IN, D_H), lambda i: (0, 0)),
        ],
        out_specs=pl.BlockSpec((RB, D_H), lambda i: (i, 0)),
        out_shape=jax.ShapeDtypeStruct((N, D_H), jnp.float32),
    )(x, W1)


def _scale_body(xw_ref, degp_ref, ylo_ref, yhi_ref, dinv_ref):
    deg = degp_ref[0][:, 0:1] + degp_ref[1][:, 0:1] + 1.0
    dinv = lax.rsqrt(deg)
    y = xw_ref[...] * dinv
    ylo_ref[...] = y[:, :HALF]
    yhi_ref[...] = y[:, HALF:]
    dinv_ref[...] = dinv


def _first_layer(xw, degp):
    return pl.pallas_call(
        _scale_body,
        grid=(GRID,),
        in_specs=[
            pl.BlockSpec((RB, D_H), lambda i: (i, 0)),
            pl.BlockSpec((2, RB, HALF), lambda i: (0, i, 0)),
        ],
        out_specs=[
            pl.BlockSpec((RB, HALF), lambda i: (i, 0)),
            pl.BlockSpec((RB, HALF), lambda i: (i, 0)),
            pl.BlockSpec((RB, 1), lambda i: (i, 0)),
        ],
        out_shape=[
            jax.ShapeDtypeStruct((N, HALF), jnp.float32),
            jax.ShapeDtypeStruct((N, HALF), jnp.float32),
            jax.ShapeDtypeStruct((N, 1), jnp.float32),
        ],
    )(xw, degp)


def _mid_body(alo_ref, ahi_ref, ylo_ref, yhi_ref, dinv_ref, w_ref, b_ref,
              olo_ref, ohi_ref):
    dinv = dinv_ref[...]
    agg = jnp.concatenate([alo_ref[...] + ylo_ref[...],
                           ahi_ref[...] + yhi_ref[...]], axis=1)
    h = jnp.maximum(agg * dinv + b_ref[...], 0.0)
    y = jnp.dot(h, w_ref[...], preferred_element_type=jnp.float32) * dinv
    olo_ref[...] = y[:, :HALF]
    ohi_ref[...] = y[:, HALF:]


def _mid_layer(alo, ahi, ylo, yhi, dinv, W, b):
    return pl.pallas_call(
        _mid_body,
        grid=(GRID,),
        in_specs=[
            pl.BlockSpec((RB, HALF), lambda i: (i, 0)),
            pl.BlockSpec((RB, HALF), lambda i: (i, 0)),
            pl.BlockSpec((RB, HALF), lambda i: (i, 0)),
            pl.BlockSpec((RB, HALF), lambda i: (i, 0)),
            pl.BlockSpec((RB, 1), lambda i: (i, 0)),
            pl.BlockSpec((D_H, D_H), lambda i: (0, 0)),
            pl.BlockSpec((1, D_H), lambda i: (0, 0)),
        ],
        out_specs=[
            pl.BlockSpec((RB, HALF), lambda i: (i, 0)),
            pl.BlockSpec((RB, HALF), lambda i: (i, 0)),
        ],
        out_shape=[
            jax.ShapeDtypeStruct((N, HALF), jnp.float32),
            jax.ShapeDtypeStruct((N, HALF), jnp.float32),
        ],
    )(alo, ahi, ylo, yhi, dinv, W, b.reshape(1, D_H))


def _head_body(alo_ref, ahi_ref, ylo_ref, yhi_ref, dinv_ref, b3_ref,
               fc1w_ref, fc1b_ref, fc2w_ref, fc2b_ref, out_ref):
    dinv = dinv_ref[...]
    agg = jnp.concatenate([alo_ref[...] + ylo_ref[...],
                           ahi_ref[...] + yhi_ref[...]], axis=1)
    h = jnp.maximum(agg * dinv + b3_ref[...], 0.0)
    z = jnp.maximum(
        jnp.dot(h, fc1w_ref[...], preferred_element_type=jnp.float32)
        + fc1b_ref[...], 0.0)
    out_ref[...] = (
        jnp.dot(z, fc2w_ref[...], preferred_element_type=jnp.float32)
        + fc2b_ref[...])


def _head(alo, ahi, ylo, yhi, dinv, b3, fc1_w, fc1_b, fc2_w, fc2_b):
    return pl.pallas_call(
        _head_body,
        grid=(GRID,),
        in_specs=[
            pl.BlockSpec((RB, HALF), lambda i: (i, 0)),
            pl.BlockSpec((RB, HALF), lambda i: (i, 0)),
            pl.BlockSpec((RB, HALF), lambda i: (i, 0)),
            pl.BlockSpec((RB, HALF), lambda i: (i, 0)),
            pl.BlockSpec((RB, 1), lambda i: (i, 0)),
            pl.BlockSpec((1, D_H), lambda i: (0, 0)),
            pl.BlockSpec((D_H, D_H // 2), lambda i: (0, 0)),
            pl.BlockSpec((1, D_H // 2), lambda i: (0, 0)),
            pl.BlockSpec((D_H // 2, 1), lambda i: (0, 0)),
            pl.BlockSpec((1, 1), lambda i: (0, 0)),
        ],
        out_specs=pl.BlockSpec((RB, 1), lambda i: (i, 0)),
        out_shape=jax.ShapeDtypeStruct((N, 1), jnp.float32),
    )(alo, ahi, ylo, yhi, dinv, b3.reshape(1, D_H), fc1_w,
      fc1_b.reshape(1, D_H // 2), fc2_w, fc2_b.reshape(1, 1))


def kernel(x, edge_index, W1, b1, W2, b2, W3, b3, fc1_w, fc1_b, fc2_w, fc2_b):
    src = edge_index[0].astype(jnp.int32)
    dst = edge_index[1].astype(jnp.int32)
    npad = EPAD - E
    pad_idx = jnp.arange(npad, dtype=jnp.int32)
    src_p = jnp.concatenate([src, pad_idx % N])
    dst_p = jnp.concatenate([dst, N + (pad_idx % L)])
    gidx = src_p.reshape(EPAD // CH, CH)
    dst2d = dst_p.reshape(EPAD // CH, CH)

    ones_deg = jnp.ones((CH, HALF), jnp.float32)
    zeros_agg = jnp.zeros((NPAD, HALF), jnp.float32)

    degp = _deg_kernel(dst2d, ones_deg, zeros_agg)
    degp = degp.reshape(NC, NPAD, HALF)

    xw1 = _xw_first(x, W1)
    y1lo, y1hi, dinv = _first_layer(xw1, degp)
    a1lo, a1hi = _agg_kernel(y1lo, y1hi, gidx, dst2d, zeros_agg)
    y2lo, y2hi = _mid_layer(a1lo, a1hi, y1lo, y1hi, dinv, W2, b1)
    a2lo, a2hi = _agg_kernel(y2lo, y2hi, gidx, dst2d, zeros_agg)
    y3lo, y3hi = _mid_layer(a2lo, a2hi, y2lo, y2hi, dinv, W3, b2)
    a3lo, a3hi = _agg_kernel(y3lo, y3hi, gidx, dst2d, zeros_agg)
    out = _head(a3lo, a3hi, y3lo, y3hi, dinv, b3,
                fc1_w, fc1_b, fc2_w, fc2_b)
    return out
